# baseline (speedup 1.0000x reference)
import math

import jax
import jax.numpy as jnp
from jax import lax
from jax.experimental import pallas as pl

_NUM_CLASSES = 80
_REG_MAX = 16
_TOPK = 13
_EPS = 1e-9
_CEPS = 1e-7
_FOUR_OVER_PI2 = 4.0 / math.pi**2
_TAN_3PI_8 = 2.414213562373095
_TAN_PI_8 = 0.4142135623730950


def _atan(x):
    c1 = x > _TAN_3PI_8
    c2 = x > _TAN_PI_8
    xr = jnp.where(c1, -1.0 / jnp.maximum(x, 1e-30),
                   jnp.where(c2, (x - 1.0) / (x + 1.0), x))
    off = jnp.where(c1, math.pi / 2.0, jnp.where(c2, math.pi / 4.0, 0.0))
    z = xr * xr
    p = ((((8.05374449538e-2 * z - 1.38776856032e-1) * z
           + 1.99777106478e-1) * z - 3.33329491539e-1) * z * xr + xr)
    return off + p


def _loss_kernel(ps_ref, pdis_ref, pbt_ref, anct_ref, gtb_ref, gtc_ref,
                 gtm_ref, out_ref):
    A = ps_ref.shape[1]
    C = ps_ref.shape[2]
    M = gtb_ref.shape[1]
    D4 = pdis_ref.shape[2]

    x = ps_ref[0]
    xd = pdis_ref[0]
    pb = pbt_ref[0]
    anc = anct_ref[...]
    g = gtb_ref[0]
    cls_col = gtc_ref[0]
    gmask_f = gtm_ref[0]

    ax = anc[0:1, :]
    ay = anc[1:2, :]
    x1p = pb[0:1, :]
    y1p = pb[1:2, :]
    x2p = pb[2:3, :]
    y2p = pb[3:4, :]
    x1g = g[:, 0:1]
    y1g = g[:, 1:2]
    x2g = g[:, 2:3]
    y2g = g[:, 3:4]

    amask = (((ax - x1g) > 1e-9) & ((ay - y1g) > 1e-9)
             & ((x2g - ax) > 1e-9) & ((y2g - ay) > 1e-9))
    amask_f = jnp.where(amask, 1.0, 0.0)

    iota_mc = lax.broadcasted_iota(jnp.int32, (M, C), 1)
    onehot_g = jnp.where(iota_mc == cls_col, 1.0, 0.0)
    logit_ma = lax.dot_general(onehot_g, x, (((1,), (1,)), ((), ())),
                               preferred_element_type=jnp.float32)
    scores = jax.nn.sigmoid(logit_ma) * amask_f

    w1 = x2g - x1g
    h1 = y2g - y1g
    w2 = x2p - x1p
    h2 = y2p - y1p
    iw = jnp.maximum(jnp.minimum(x2g, x2p) - jnp.maximum(x1g, x1p), 0.0)
    ih = jnp.maximum(jnp.minimum(y2g, y2p) - jnp.maximum(y1g, y1p), 0.0)
    inter = iw * ih
    union = w1 * h1 + w2 * h2 - inter + _CEPS
    iou = inter / union
    cw = jnp.maximum(x2g, x2p) - jnp.minimum(x1g, x1p)
    ch = jnp.maximum(y2g, y2p) - jnp.minimum(y1g, y1p)
    c2 = cw * cw + ch * ch + _CEPS
    dx = x1p + x2p - x1g - x2g
    dy = y1p + y2p - y1g - y2g
    rho2 = (dx * dx + dy * dy) * 0.25
    atan_pd = _atan(w2 / (h2 + _CEPS))
    atan_gt = _atan(w1 / (h1 + _CEPS))
    dv = atan_pd - atan_gt
    v = _FOUR_OVER_PI2 * dv * dv
    alpha = v / (v - iou + (1.0 + _CEPS))
    ciou = iou - (rho2 / c2 + v * alpha)
    ious = jnp.maximum(ciou, 0.0) * amask_f

    i2 = ious * ious
    align = scores * (i2 * i2 * i2)

    iota_a = lax.broadcasted_iota(jnp.int32, (M, A), 1)
    vals = align
    for _ in range(_TOPK):
        rmax = jnp.max(vals, axis=1, keepdims=True)
        cand = jnp.where(vals == rmax, iota_a, A)
        imin = jnp.min(cand, axis=1, keepdims=True)
        vals = jnp.where(cand == imin, -1.0, vals)
    sel_f = jnp.where(vals < 0.0, 1.0, 0.0)
    sel_f = sel_f * gmask_f
    pos_f = amask_f * sel_f

    fg_count = jnp.sum(pos_f, axis=0, keepdims=True)
    iota_m = lax.broadcasted_iota(jnp.int32, (M, A), 0)
    cmax = jnp.max(ious, axis=0, keepdims=True)
    candm = jnp.where(ious == cmax, iota_m, M)
    am = jnp.min(candm, axis=0, keepdims=True)
    is_max_f = jnp.where(iota_m == am, 1.0, 0.0)
    pos_f = jnp.where(fg_count > 1.0, is_max_f, pos_f)
    fg_f = jnp.max(pos_f, axis=0, keepdims=True)

    candt = jnp.where(pos_f > 0.0, iota_m, M)
    tmin = jnp.min(candt, axis=0, keepdims=True)
    tidx = jnp.where(tmin == M, 0, tmin)
    onehot_t = jnp.where(iota_m == tidx, 1.0, 0.0)
    tbx1 = jnp.sum(onehot_t * x1g, axis=0, keepdims=True)
    tby1 = jnp.sum(onehot_t * y1g, axis=0, keepdims=True)
    tbx2 = jnp.sum(onehot_t * x2g, axis=0, keepdims=True)
    tby2 = jnp.sum(onehot_t * y2g, axis=0, keepdims=True)

    align_p = align * pos_f
    max_align = jnp.max(align_p, axis=1, keepdims=True)
    max_ious = jnp.max(ious * pos_f, axis=1, keepdims=True)
    normc = align_p * (max_ious / (max_align + _EPS))
    norm = jnp.max(normc, axis=0, keepdims=True)
    weight = norm * fg_f
    p4 = jnp.sum(weight)

    p0 = jnp.sum(jnp.maximum(x, 0.0) + jnp.log(1.0 + jnp.exp(-jnp.abs(x))))
    p1 = jnp.sum(onehot_t * logit_ma * weight)

    tw = tbx2 - tbx1
    th = tby2 - tby1
    iw2 = jnp.maximum(jnp.minimum(x2p, tbx2) - jnp.maximum(x1p, tbx1), 0.0)
    ih2 = jnp.maximum(jnp.minimum(y2p, tby2) - jnp.maximum(y1p, tby1), 0.0)
    inter2 = iw2 * ih2
    union2 = w2 * h2 + tw * th - inter2 + _CEPS
    iou2 = inter2 / union2
    cw2 = jnp.maximum(x2p, tbx2) - jnp.minimum(x1p, tbx1)
    ch2 = jnp.maximum(y2p, tby2) - jnp.minimum(y1p, tby1)
    c22 = cw2 * cw2 + ch2 * ch2 + _CEPS
    dx2 = tbx1 + tbx2 - x1p - x2p
    dy2 = tby1 + tby2 - y1p - y2p
    rho22 = (dx2 * dx2 + dy2 * dy2) * 0.25
    atan_t = _atan(tw / (th + _CEPS))
    dv2 = atan_t - atan_pd
    v2 = _FOUR_OVER_PI2 * dv2 * dv2
    alpha2 = v2 / (v2 - iou2 + (1.0 + _CEPS))
    ciou2 = iou2 - (rho22 / c22 + v2 * alpha2)
    p2 = jnp.sum((1.0 - ciou2) * weight)

    t0 = ax - tbx1
    t1 = ay - tby1
    t2 = tbx2 - ax
    t3 = tby2 - ay
    E = jnp.exp(xd)
    iota_r = lax.broadcasted_iota(jnp.int32, (D4, D4), 0)
    iota_c = lax.broadcasted_iota(jnp.int32, (D4, D4), 1)
    S = jnp.where((iota_r >> 4) == (iota_c >> 4), 1.0, 0.0)
    G = lax.dot_general(E, S, (((1,), (0,)), ((), ())),
                        preferred_element_type=jnp.float32)
    L = jnp.log(G) - xd
    R = _REG_MAX
    tsel = jnp.concatenate([jnp.broadcast_to(t, (R, A))
                            for t in (t0, t1, t2, t3)], axis=0)
    tsel = jnp.clip(tsel, 0.0, _REG_MAX - 1 - 0.01)
    si = lax.broadcasted_iota(jnp.int32, (D4, A), 0)
    binf = (si & 15).astype(jnp.float32)
    wmap = jnp.maximum(1.0 - jnp.abs(binf - tsel), 0.0)
    wt = wmap * (weight * 0.25)
    Z = lax.dot_general(wt, L, (((1,), (0,)), ((), ())),
                        preferred_element_type=jnp.float32)
    eye_d = (iota_r == iota_c)
    p3 = jnp.sum(jnp.where(eye_d, Z, 0.0))

    iota8 = lax.broadcasted_iota(jnp.int32, (1, 8), 1)
    vec = (jnp.where(iota8 == 0, p0, 0.0) + jnp.where(iota8 == 1, p1, 0.0)
           + jnp.where(iota8 == 2, p2, 0.0) + jnp.where(iota8 == 3, p3, 0.0)
           + jnp.where(iota8 == 4, p4, 0.0))
    out_ref[0] = vec


def _run(pd_scores, pd_distri, pbt, anct, gt_boxes, gtc, gtm_f,
         interpret=False):
    Bb, Aa, Cc = pd_scores.shape
    Mm = gt_boxes.shape[1]
    D4 = pd_distri.shape[2]
    return pl.pallas_call(
        _loss_kernel,
        grid=(Bb,),
        in_specs=[
            pl.BlockSpec((1, Aa, Cc), lambda b: (b, 0, 0)),
            pl.BlockSpec((1, Aa, D4), lambda b: (b, 0, 0)),
            pl.BlockSpec((1, 4, Aa), lambda b: (b, 0, 0)),
            pl.BlockSpec((2, Aa), lambda b: (0, 0)),
            pl.BlockSpec((1, Mm, 4), lambda b: (b, 0, 0)),
            pl.BlockSpec((1, Mm, 1), lambda b: (b, 0, 0)),
            pl.BlockSpec((1, Mm, 1), lambda b: (b, 0, 0)),
        ],
        out_specs=pl.BlockSpec((1, 1, 8), lambda b: (b, 0, 0)),
        out_shape=jax.ShapeDtypeStruct((Bb, 1, 8), jnp.float32),
        interpret=interpret,
    )(pd_scores, pd_distri, pbt, anct, gt_boxes, gtc, gtm_f)


def kernel(pd_scores, pd_distri, pd_boxes, anchor_tensor, gt_classes,
           gt_boxes, gt_mask):
    pbt = jnp.transpose(pd_boxes, (0, 2, 1))
    anct = jnp.transpose(anchor_tensor, (1, 0))
    gtm_f = gt_mask.astype(jnp.float32)
    parts = _run(pd_scores, pd_distri, pbt, anct, gt_boxes,
                 gt_classes.astype(jnp.int32), gtm_f)
    s = jnp.sum(parts, axis=(0, 1))
    tss = jnp.maximum(s[4], 1.0)
    loss_cls = (s[0] - s[1]) / tss
    loss_iou = s[2] / tss
    loss_dfl = s[3] / tss
    return 7.5 * loss_iou + 0.5 * loss_cls + 1.5 * loss_dfl


# speedup vs baseline: 1.0691x; 1.0691x over previous
import math

import jax
import jax.numpy as jnp
from jax import lax
from jax.experimental import pallas as pl

_NUM_CLASSES = 80
_REG_MAX = 16
_TOPK = 13
_EPS = 1e-9
_CEPS = 1e-7
_FOUR_OVER_PI2 = 4.0 / math.pi**2
_TAN_3PI_8 = 2.414213562373095
_TAN_PI_8 = 0.4142135623730950


def _atan(x):
    c1 = x > _TAN_3PI_8
    c2 = x > _TAN_PI_8
    xr = jnp.where(c1, -1.0 / jnp.maximum(x, 1e-30),
                   jnp.where(c2, (x - 1.0) / (x + 1.0), x))
    off = jnp.where(c1, math.pi / 2.0, jnp.where(c2, math.pi / 4.0, 0.0))
    z = xr * xr
    p = ((((8.05374449538e-2 * z - 1.38776856032e-1) * z
           + 1.99777106478e-1) * z - 3.33329491539e-1) * z * xr + xr)
    return off + p


def _loss_kernel(ps_ref, pdis_ref, pbt_ref, anct_ref, gtb_ref, gtc_ref,
                 gtm_ref, out_ref):
    A = ps_ref.shape[1]
    C = ps_ref.shape[2]
    M = gtb_ref.shape[1]
    D4 = pdis_ref.shape[2]

    x = ps_ref[0]
    xd = pdis_ref[0]
    pb = pbt_ref[0]
    anc = anct_ref[...]
    g = gtb_ref[0]
    cls_col = gtc_ref[0]
    gmask_f = gtm_ref[0]

    ax = anc[0:1, :]
    ay = anc[1:2, :]
    x1p = pb[0:1, :]
    y1p = pb[1:2, :]
    x2p = pb[2:3, :]
    y2p = pb[3:4, :]
    x1g = g[:, 0:1]
    y1g = g[:, 1:2]
    x2g = g[:, 2:3]
    y2g = g[:, 3:4]

    amask = (((ax - x1g) > 1e-9) & ((ay - y1g) > 1e-9)
             & ((x2g - ax) > 1e-9) & ((y2g - ay) > 1e-9))
    amask_f = jnp.where(amask, 1.0, 0.0)

    iota_mc = lax.broadcasted_iota(jnp.int32, (M, C), 1)
    onehot_g = jnp.where(iota_mc == cls_col, 1.0, 0.0)
    logit_ma = lax.dot_general(onehot_g, x, (((1,), (1,)), ((), ())),
                               preferred_element_type=jnp.float32)
    scores = jax.nn.sigmoid(logit_ma)

    w1 = x2g - x1g
    h1 = y2g - y1g
    w2 = x2p - x1p
    h2 = y2p - y1p
    iw = jnp.maximum(jnp.minimum(x2g, x2p) - jnp.maximum(x1g, x1p), 0.0)
    ih = jnp.maximum(jnp.minimum(y2g, y2p) - jnp.maximum(y1g, y1p), 0.0)
    inter = iw * ih
    union = w1 * h1 + w2 * h2 - inter + _CEPS
    iou = inter / union
    cw = jnp.maximum(x2g, x2p) - jnp.minimum(x1g, x1p)
    ch = jnp.maximum(y2g, y2p) - jnp.minimum(y1g, y1p)
    c2 = cw * cw + ch * ch + _CEPS
    dx = x1p + x2p - x1g - x2g
    dy = y1p + y2p - y1g - y2g
    rho2 = (dx * dx + dy * dy) * 0.25
    atan_pd = _atan(w2 / (h2 + _CEPS))
    atan_gt = _atan(w1 / (h1 + _CEPS))
    dv = atan_pd - atan_gt
    v = _FOUR_OVER_PI2 * dv * dv
    alpha = v / (v - iou + (1.0 + _CEPS))
    ciou = iou - (rho2 / c2 + v * alpha)
    ious = jnp.maximum(ciou, 0.0) * amask_f

    i2 = ious * ious
    align = scores * (i2 * i2 * i2)

    iota_af = lax.broadcasted_iota(jnp.int32, (M, A), 1).astype(jnp.float32)
    a_sent = jnp.float32(A)
    vals = align
    for _ in range(_TOPK):
        rmax = jnp.max(vals, axis=1, keepdims=True)
        cand = jnp.where(vals == rmax, iota_af, a_sent)
        imin = jnp.min(cand, axis=1, keepdims=True)
        vals = jnp.where(cand == imin, -1.0, vals)
    sel_f = jnp.where(vals < 0.0, 1.0, 0.0)
    sel_f = sel_f * gmask_f
    pos_f = amask_f * sel_f

    fg_count = jnp.sum(pos_f, axis=0, keepdims=True)
    iota_m = lax.broadcasted_iota(jnp.int32, (M, A), 0)
    cmax = jnp.max(ious, axis=0, keepdims=True)
    candm = jnp.where(ious == cmax, iota_m, M)
    am = jnp.min(candm, axis=0, keepdims=True)
    is_max_f = jnp.where(iota_m == am, 1.0, 0.0)
    pos_f = jnp.where(fg_count > 1.0, is_max_f, pos_f)
    fg_f = jnp.max(pos_f, axis=0, keepdims=True)

    candt = jnp.where(pos_f > 0.0, iota_m, M)
    tmin = jnp.min(candt, axis=0, keepdims=True)
    tidx = jnp.where(tmin == M, 0, tmin)
    onehot_t = jnp.where(iota_m == tidx, 1.0, 0.0)
    tbx1 = jnp.sum(onehot_t * x1g, axis=0, keepdims=True)
    tby1 = jnp.sum(onehot_t * y1g, axis=0, keepdims=True)
    tbx2 = jnp.sum(onehot_t * x2g, axis=0, keepdims=True)
    tby2 = jnp.sum(onehot_t * y2g, axis=0, keepdims=True)

    align_p = align * pos_f
    max_align = jnp.max(align_p, axis=1, keepdims=True)
    max_ious = jnp.max(ious * pos_f, axis=1, keepdims=True)
    normc = align_p * (max_ious / (max_align + _EPS))
    norm = jnp.max(normc, axis=0, keepdims=True)
    weight = norm * fg_f
    p4 = jnp.sum(weight)

    p0 = jnp.sum(jnp.maximum(x, 0.0) + jnp.log(1.0 + jnp.exp(-jnp.abs(x))))
    p1 = jnp.sum(onehot_t * logit_ma * weight)

    tw = tbx2 - tbx1
    th = tby2 - tby1
    iw2 = jnp.maximum(jnp.minimum(x2p, tbx2) - jnp.maximum(x1p, tbx1), 0.0)
    ih2 = jnp.maximum(jnp.minimum(y2p, tby2) - jnp.maximum(y1p, tby1), 0.0)
    inter2 = iw2 * ih2
    union2 = w2 * h2 + tw * th - inter2 + _CEPS
    iou2 = inter2 / union2
    cw2 = jnp.maximum(x2p, tbx2) - jnp.minimum(x1p, tbx1)
    ch2 = jnp.maximum(y2p, tby2) - jnp.minimum(y1p, tby1)
    c22 = cw2 * cw2 + ch2 * ch2 + _CEPS
    dx2 = tbx1 + tbx2 - x1p - x2p
    dy2 = tby1 + tby2 - y1p - y2p
    rho22 = (dx2 * dx2 + dy2 * dy2) * 0.25
    atan_t = _atan(tw / (th + _CEPS))
    dv2 = atan_t - atan_pd
    v2 = _FOUR_OVER_PI2 * dv2 * dv2
    alpha2 = v2 / (v2 - iou2 + (1.0 + _CEPS))
    ciou2 = iou2 - (rho22 / c22 + v2 * alpha2)
    p2 = jnp.sum((1.0 - ciou2) * weight)

    t0 = ax - tbx1
    t1 = ay - tby1
    t2 = tbx2 - ax
    t3 = tby2 - ay
    E = jnp.exp(xd)
    iota_r = lax.broadcasted_iota(jnp.int32, (D4, D4), 0)
    iota_c = lax.broadcasted_iota(jnp.int32, (D4, D4), 1)
    S = jnp.where((iota_r >> 4) == (iota_c >> 4), 1.0, 0.0)
    G = lax.dot_general(E, S, (((1,), (0,)), ((), ())),
                        preferred_element_type=jnp.float32)
    L = jnp.log(G) - xd
    R = _REG_MAX
    tsel = jnp.concatenate([jnp.broadcast_to(t, (R, A))
                            for t in (t0, t1, t2, t3)], axis=0)
    tsel = jnp.clip(tsel, 0.0, _REG_MAX - 1 - 0.01)
    si = lax.broadcasted_iota(jnp.int32, (D4, A), 0)
    binf = (si & 15).astype(jnp.float32)
    wmap = jnp.maximum(1.0 - jnp.abs(binf - tsel), 0.0)
    wt = wmap * (weight * 0.25)
    Z = lax.dot_general(wt, L, (((1,), (0,)), ((), ())),
                        preferred_element_type=jnp.float32)
    eye_d = (iota_r == iota_c)
    p3 = jnp.sum(jnp.where(eye_d, Z, 0.0))

    iota8 = lax.broadcasted_iota(jnp.int32, (1, 8), 1)
    vec = (jnp.where(iota8 == 0, p0, 0.0) + jnp.where(iota8 == 1, p1, 0.0)
           + jnp.where(iota8 == 2, p2, 0.0) + jnp.where(iota8 == 3, p3, 0.0)
           + jnp.where(iota8 == 4, p4, 0.0))
    out_ref[0] = vec


def _run(pd_scores, pd_distri, pbt, anct, gt_boxes, gtc, gtm_f,
         interpret=False):
    Bb, Aa, Cc = pd_scores.shape
    Mm = gt_boxes.shape[1]
    D4 = pd_distri.shape[2]
    return pl.pallas_call(
        _loss_kernel,
        grid=(Bb,),
        in_specs=[
            pl.BlockSpec((1, Aa, Cc), lambda b: (b, 0, 0)),
            pl.BlockSpec((1, Aa, D4), lambda b: (b, 0, 0)),
            pl.BlockSpec((1, 4, Aa), lambda b: (b, 0, 0)),
            pl.BlockSpec((2, Aa), lambda b: (0, 0)),
            pl.BlockSpec((1, Mm, 4), lambda b: (b, 0, 0)),
            pl.BlockSpec((1, Mm, 1), lambda b: (b, 0, 0)),
            pl.BlockSpec((1, Mm, 1), lambda b: (b, 0, 0)),
        ],
        out_specs=pl.BlockSpec((1, 1, 8), lambda b: (b, 0, 0)),
        out_shape=jax.ShapeDtypeStruct((Bb, 1, 8), jnp.float32),
        interpret=interpret,
    )(pd_scores, pd_distri, pbt, anct, gt_boxes, gtc, gtm_f)


def kernel(pd_scores, pd_distri, pd_boxes, anchor_tensor, gt_classes,
           gt_boxes, gt_mask):
    pbt = jnp.transpose(pd_boxes, (0, 2, 1))
    anct = jnp.transpose(anchor_tensor, (1, 0))
    gtm_f = gt_mask.astype(jnp.float32)
    parts = _run(pd_scores, pd_distri, pbt, anct, gt_boxes,
                 gt_classes.astype(jnp.int32), gtm_f)
    s = jnp.sum(parts, axis=(0, 1))
    tss = jnp.maximum(s[4], 1.0)
    loss_cls = (s[0] - s[1]) / tss
    loss_iou = s[2] / tss
    loss_dfl = s[3] / tss
    return 7.5 * loss_iou + 0.5 * loss_cls + 1.5 * loss_dfl


# speedup vs baseline: 1.0745x; 1.0051x over previous
import math

import jax
import jax.numpy as jnp
from jax import lax
from jax.experimental import pallas as pl

_NUM_CLASSES = 80
_REG_MAX = 16
_TOPK = 13
_EPS = 1e-9
_CEPS = 1e-7
_FOUR_OVER_PI2 = 4.0 / math.pi**2
_TAN_3PI_8 = 2.414213562373095
_TAN_PI_8 = 0.4142135623730950


def _atan(x):
    c1 = x > _TAN_3PI_8
    c2 = x > _TAN_PI_8
    xr = jnp.where(c1, -1.0 / jnp.maximum(x, 1e-30),
                   jnp.where(c2, (x - 1.0) / (x + 1.0), x))
    off = jnp.where(c1, math.pi / 2.0, jnp.where(c2, math.pi / 4.0, 0.0))
    z = xr * xr
    p = ((((8.05374449538e-2 * z - 1.38776856032e-1) * z
           + 1.99777106478e-1) * z - 3.33329491539e-1) * z * xr + xr)
    return off + p


def _loss_kernel(ps_ref, pdis_ref, pbt_ref, anct_ref, gtb_ref, gtc_ref,
                 gtm_ref, out_ref):
    A = ps_ref.shape[1]
    C = ps_ref.shape[2]
    M = gtb_ref.shape[1]
    D4 = pdis_ref.shape[2]

    x = ps_ref[0]
    xd = pdis_ref[0]
    pb = pbt_ref[0]
    anc = anct_ref[...]
    g = gtb_ref[0]
    cls_col = gtc_ref[0]
    gmask_f = gtm_ref[0]

    ax = anc[0:1, :]
    ay = anc[1:2, :]
    x1p = pb[0:1, :]
    y1p = pb[1:2, :]
    x2p = pb[2:3, :]
    y2p = pb[3:4, :]
    x1g = g[:, 0:1]
    y1g = g[:, 1:2]
    x2g = g[:, 2:3]
    y2g = g[:, 3:4]

    amask = (((ax - x1g) > 1e-9) & ((ay - y1g) > 1e-9)
             & ((x2g - ax) > 1e-9) & ((y2g - ay) > 1e-9))
    amask_f = jnp.where(amask, 1.0, 0.0)

    iota_mc = lax.broadcasted_iota(jnp.int32, (M, C), 1)
    onehot_g = jnp.where(iota_mc == cls_col, 1.0, 0.0)
    logit_ma = lax.dot_general(onehot_g, x, (((1,), (1,)), ((), ())),
                               preferred_element_type=jnp.float32)
    scores = jax.nn.sigmoid(logit_ma)

    w1 = x2g - x1g
    h1 = y2g - y1g
    w2 = x2p - x1p
    h2 = y2p - y1p
    iw = jnp.maximum(jnp.minimum(x2g, x2p) - jnp.maximum(x1g, x1p), 0.0)
    ih = jnp.maximum(jnp.minimum(y2g, y2p) - jnp.maximum(y1g, y1p), 0.0)
    inter = iw * ih
    union = w1 * h1 + w2 * h2 - inter + _CEPS
    iou = inter / union
    cw = jnp.maximum(x2g, x2p) - jnp.minimum(x1g, x1p)
    ch = jnp.maximum(y2g, y2p) - jnp.minimum(y1g, y1p)
    c2 = cw * cw + ch * ch + _CEPS
    dx = x1p + x2p - x1g - x2g
    dy = y1p + y2p - y1g - y2g
    rho2 = (dx * dx + dy * dy) * 0.25
    atan_pd = _atan(w2 / (h2 + _CEPS))
    atan_gt = _atan(w1 / (h1 + _CEPS))
    dv = atan_pd - atan_gt
    v = _FOUR_OVER_PI2 * dv * dv
    alpha = v / (v - iou + (1.0 + _CEPS))
    ciou = iou - (rho2 / c2 + v * alpha)
    ious = jnp.maximum(ciou, 0.0) * amask_f

    i2 = ious * ious
    align = scores * (i2 * i2 * i2)

    iota_af = lax.broadcasted_iota(jnp.int32, (M, A), 1).astype(jnp.float32)
    a_sent = jnp.float32(A)
    vals = align
    for _ in range(_TOPK):
        rmax = jnp.max(vals, axis=1, keepdims=True)
        cand = jnp.where(vals == rmax, iota_af, a_sent)
        imin = jnp.min(cand, axis=1, keepdims=True)
        vals = jnp.where(cand == imin, -1.0, vals)
    sel_f = jnp.where(vals < 0.0, 1.0, 0.0)
    sel_f = sel_f * gmask_f
    pos_f = amask_f * sel_f

    fg_count = jnp.sum(pos_f, axis=0, keepdims=True)
    iota_mf = lax.broadcasted_iota(jnp.int32, (M, A), 0).astype(jnp.float32)
    m_sent = jnp.float32(M)
    cmax = jnp.max(ious, axis=0, keepdims=True)
    candm = jnp.where(ious == cmax, iota_mf, m_sent)
    am = jnp.min(candm, axis=0, keepdims=True)
    is_max_f = jnp.where(iota_mf == am, 1.0, 0.0)
    pos_f = jnp.where(fg_count > 1.0, is_max_f, pos_f)
    fg_f = jnp.max(pos_f, axis=0, keepdims=True)

    candt = jnp.where(pos_f > 0.0, iota_mf, m_sent)
    tmin = jnp.min(candt, axis=0, keepdims=True)
    tidx = jnp.where(tmin == m_sent, 0.0, tmin)
    onehot_t = jnp.where(iota_mf == tidx, 1.0, 0.0)
    tbx1 = jnp.sum(onehot_t * x1g, axis=0, keepdims=True)
    tby1 = jnp.sum(onehot_t * y1g, axis=0, keepdims=True)
    tbx2 = jnp.sum(onehot_t * x2g, axis=0, keepdims=True)
    tby2 = jnp.sum(onehot_t * y2g, axis=0, keepdims=True)

    align_p = align * pos_f
    max_align = jnp.max(align_p, axis=1, keepdims=True)
    max_ious = jnp.max(ious * pos_f, axis=1, keepdims=True)
    normc = align_p * (max_ious / (max_align + _EPS))
    norm = jnp.max(normc, axis=0, keepdims=True)
    weight = norm * fg_f
    p4 = jnp.sum(weight)

    p0 = jnp.sum(jnp.maximum(x, 0.0) + jnp.log(1.0 + jnp.exp(-jnp.abs(x))))
    p1 = jnp.sum(onehot_t * logit_ma * weight)

    tw = tbx2 - tbx1
    th = tby2 - tby1
    iw2 = jnp.maximum(jnp.minimum(x2p, tbx2) - jnp.maximum(x1p, tbx1), 0.0)
    ih2 = jnp.maximum(jnp.minimum(y2p, tby2) - jnp.maximum(y1p, tby1), 0.0)
    inter2 = iw2 * ih2
    union2 = w2 * h2 + tw * th - inter2 + _CEPS
    iou2 = inter2 / union2
    cw2 = jnp.maximum(x2p, tbx2) - jnp.minimum(x1p, tbx1)
    ch2 = jnp.maximum(y2p, tby2) - jnp.minimum(y1p, tby1)
    c22 = cw2 * cw2 + ch2 * ch2 + _CEPS
    dx2 = tbx1 + tbx2 - x1p - x2p
    dy2 = tby1 + tby2 - y1p - y2p
    rho22 = (dx2 * dx2 + dy2 * dy2) * 0.25
    atan_t = _atan(tw / (th + _CEPS))
    dv2 = atan_t - atan_pd
    v2 = _FOUR_OVER_PI2 * dv2 * dv2
    alpha2 = v2 / (v2 - iou2 + (1.0 + _CEPS))
    ciou2 = iou2 - (rho22 / c22 + v2 * alpha2)
    p2 = jnp.sum((1.0 - ciou2) * weight)

    t0 = ax - tbx1
    t1 = ay - tby1
    t2 = tbx2 - ax
    t3 = tby2 - ay
    E = jnp.exp(xd)
    iota_r = lax.broadcasted_iota(jnp.int32, (D4, D4), 0)
    iota_c = lax.broadcasted_iota(jnp.int32, (D4, D4), 1)
    S = jnp.where((iota_r >> 4) == (iota_c >> 4), 1.0, 0.0)
    G = lax.dot_general(E, S, (((1,), (0,)), ((), ())),
                        preferred_element_type=jnp.float32)
    L = jnp.log(G) - xd
    R = _REG_MAX
    tsel = jnp.concatenate([jnp.broadcast_to(t, (R, A))
                            for t in (t0, t1, t2, t3)], axis=0)
    tsel = jnp.clip(tsel, 0.0, _REG_MAX - 1 - 0.01)
    si = lax.broadcasted_iota(jnp.int32, (D4, A), 0)
    binf = (si & 15).astype(jnp.float32)
    wmap = jnp.maximum(1.0 - jnp.abs(binf - tsel), 0.0)
    wt = wmap * (weight * 0.25)
    Z = lax.dot_general(wt, L, (((1,), (0,)), ((), ())),
                        preferred_element_type=jnp.float32)
    eye_d = (iota_r == iota_c)
    p3 = jnp.sum(jnp.where(eye_d, Z, 0.0))

    iota8 = lax.broadcasted_iota(jnp.int32, (1, 8), 1)
    vec = (jnp.where(iota8 == 0, p0, 0.0) + jnp.where(iota8 == 1, p1, 0.0)
           + jnp.where(iota8 == 2, p2, 0.0) + jnp.where(iota8 == 3, p3, 0.0)
           + jnp.where(iota8 == 4, p4, 0.0))
    out_ref[0] = vec


def _run(pd_scores, pd_distri, pbt, anct, gt_boxes, gtc, gtm_f,
         interpret=False):
    Bb, Aa, Cc = pd_scores.shape
    Mm = gt_boxes.shape[1]
    D4 = pd_distri.shape[2]
    return pl.pallas_call(
        _loss_kernel,
        grid=(Bb,),
        in_specs=[
            pl.BlockSpec((1, Aa, Cc), lambda b: (b, 0, 0)),
            pl.BlockSpec((1, Aa, D4), lambda b: (b, 0, 0)),
            pl.BlockSpec((1, 4, Aa), lambda b: (b, 0, 0)),
            pl.BlockSpec((2, Aa), lambda b: (0, 0)),
            pl.BlockSpec((1, Mm, 4), lambda b: (b, 0, 0)),
            pl.BlockSpec((1, Mm, 1), lambda b: (b, 0, 0)),
            pl.BlockSpec((1, Mm, 1), lambda b: (b, 0, 0)),
        ],
        out_specs=pl.BlockSpec((1, 1, 8), lambda b: (b, 0, 0)),
        out_shape=jax.ShapeDtypeStruct((Bb, 1, 8), jnp.float32),
        interpret=interpret,
    )(pd_scores, pd_distri, pbt, anct, gt_boxes, gtc, gtm_f)


def kernel(pd_scores, pd_distri, pd_boxes, anchor_tensor, gt_classes,
           gt_boxes, gt_mask):
    pbt = jnp.transpose(pd_boxes, (0, 2, 1))
    anct = jnp.transpose(anchor_tensor, (1, 0))
    gtm_f = gt_mask.astype(jnp.float32)
    parts = _run(pd_scores, pd_distri, pbt, anct, gt_boxes,
                 gt_classes.astype(jnp.int32), gtm_f)
    s = jnp.sum(parts, axis=(0, 1))
    tss = jnp.maximum(s[4], 1.0)
    loss_cls = (s[0] - s[1]) / tss
    loss_iou = s[2] / tss
    loss_dfl = s[3] / tss
    return 7.5 * loss_iou + 0.5 * loss_cls + 1.5 * loss_dfl
